# baseline (device time: 58172 ns/iter reference)
import os

import jax
import jax.numpy as jnp
from jax import lax
from jax.experimental import pallas as pl
from jax.experimental.pallas import tpu as pltpu

N_DEV = 4
K_DMA = 32

DO_ZERO = os.environ.get("KZ", "1") == "1"
DO_GATHER = os.environ.get("KG", "1") == "1"
DO_COMM = os.environ.get("KC", "1") == "1"
N_ROUNDS = int(os.environ.get("KN", "4"))


def _body(pos_ref, idx_ref, cnt_ref, e_ref, out_ref,
          bx1, bx2, by1, by2, gather_sems, xs, xr, ys, yr):
    t, d = out_ref.shape
    t2 = t // 2
    b2 = t // 4
    b4 = t // 8

    my = lax.axis_index("i")
    a = my % 2
    b = my // 2
    k1 = (a + b) % 2
    p_a = my + 1 - 2 * a
    p_b = 3 - my

    def xfer(src_rows, n_rows, dst, ssem, rsem, peer):
        return pltpu.make_async_remote_copy(
            src_ref=out_ref.at[pl.ds(src_rows, n_rows), :],
            dst_ref=dst,
            send_sem=ssem,
            recv_sem=rsem,
            device_id=(peer,),
            device_id_type=pl.DeviceIdType.MESH,
        )

    def row_dma(j):
        return pltpu.make_async_copy(
            e_ref.at[pl.ds(idx_ref[j], 1), :],
            out_ref.at[pl.ds(pos_ref[j], 1), :],
            gather_sems.at[j % K_DMA],
        )

    def gather(lo, hi):
        def step(j, carry):
            @pl.when(j - lo >= K_DMA)
            def _():
                row_dma(j - K_DMA).wait()
            row_dma(j).start()
            return carry

        lax.fori_loop(lo, hi, step, 0)

        def drain(j, carry):
            row_dma(j).wait()
            return carry

        lax.fori_loop(jnp.maximum(hi - K_DMA, lo), hi, drain, 0)

    barrier_sem = pltpu.get_barrier_semaphore()
    for nbr in [p_a, p_b]:
        pl.semaphore_signal(
            barrier_sem, inc=1,
            device_id=(nbr,), device_id_type=pl.DeviceIdType.MESH,
        )
    pl.semaphore_wait(barrier_sem, 2)

    nx = cnt_ref[0]
    ny = cnt_ref[1]
    nr = cnt_ref[2]

    x_send = (1 - k1) * b2
    y_send = t2 + (1 - b) * b2
    x_keep = k1 * b2
    y_keep = t2 + b * b2
    x_q_keep = x_keep + b * b4
    y_q_keep = y_keep + a * b4

    def zero_half(off):
        out_ref[pl.ds(off, b2), :] = jnp.zeros((b2, d), jnp.float32)

    def add_quarter(base, q, buf):
        out_ref[pl.ds(base + q, b4), :] = (
            out_ref[pl.ds(base + q, b4), :] + buf[pl.ds(q, b4), :]
        )

    if DO_ZERO:
        zero_half(x_send)
    if DO_GATHER:
        gather(0, nx)
    if not DO_COMM:
        if DO_ZERO:
            zero_half(y_send)
            zero_half(x_keep)
            zero_half(y_keep)
        if DO_GATHER:
            gather(b2, b2 + ny)
            gather(2 * b2, 2 * b2 + nr)
        return
    x1a = xfer(x_send + (1 - b) * b4, b4,
               bx1.at[pl.ds((1 - b) * b4, b4), :], xs.at[0], xr.at[0], p_a)
    x1b = xfer(x_send + b * b4, b4,
               bx1.at[pl.ds(b * b4, b4), :], xs.at[1], xr.at[1], p_a)
    x1a.start()
    x1b.start()
    if DO_ZERO:
        zero_half(y_send)
    if DO_GATHER:
        gather(b2, b2 + ny)
    y1a = xfer(y_send + a * b4, b4,
               by1.at[pl.ds(a * b4, b4), :], ys.at[0], yr.at[0], p_b)
    y1b = xfer(y_send + (1 - a) * b4, b4,
               by1.at[pl.ds((1 - a) * b4, b4), :], ys.at[1], yr.at[1], p_b)
    y1a.start()
    y1b.start()

    if DO_ZERO:
        zero_half(x_keep)
        zero_half(y_keep)
    if DO_GATHER:
        gather(2 * b2, 2 * b2 + nr)

    if N_ROUNDS <= 1:
        x1a.wait()
        x1b.wait()
        y1a.wait()
        y1b.wait()
        return

    x1a.wait()
    add_quarter(x_keep, (1 - b) * b4, bx1)
    x2 = xfer(x_keep + (1 - b) * b4, b4, bx2, xs.at[2], xr.at[2], p_b)
    x2.start()
    y1a.wait()
    add_quarter(y_keep, (1 - a) * b4, by1)
    y2 = xfer(y_keep + (1 - a) * b4, b4, by2, ys.at[2], yr.at[2], p_a)
    y2.start()
    x1b.wait()
    add_quarter(x_keep, b * b4, bx1)
    y1b.wait()
    add_quarter(y_keep, a * b4, by1)

    if N_ROUNDS <= 2:
        x2.wait()
        out_ref[pl.ds(x_q_keep, b4), :] = (
            out_ref[pl.ds(x_q_keep, b4), :] + bx2[:, :]
        )
        y2.wait()
        out_ref[pl.ds(y_q_keep, b4), :] = (
            out_ref[pl.ds(y_q_keep, b4), :] + by2[:, :]
        )
        return

    x2.wait()
    out_ref[pl.ds(x_q_keep, b4), :] = (
        out_ref[pl.ds(x_q_keep, b4), :] + bx2[:, :]
    )
    x3 = xfer(x_q_keep, b4, out_ref.at[pl.ds(x_q_keep, b4), :],
              xs.at[3], xr.at[3], p_b)
    x3.start()
    x4a = xfer(x_q_keep, b4, out_ref.at[pl.ds(x_q_keep, b4), :],
               xs.at[4], xr.at[4], p_a)
    x4a.start()
    y2.wait()
    out_ref[pl.ds(y_q_keep, b4), :] = (
        out_ref[pl.ds(y_q_keep, b4), :] + by2[:, :]
    )
    y3 = xfer(y_q_keep, b4, out_ref.at[pl.ds(y_q_keep, b4), :],
              ys.at[3], yr.at[3], p_a)
    y3.start()
    y4a = xfer(y_q_keep, b4, out_ref.at[pl.ds(y_q_keep, b4), :],
               ys.at[4], yr.at[4], p_b)
    y4a.start()

    if N_ROUNDS <= 3:
        x3.wait()
        y3.wait()
        x4a.wait()
        y4a.wait()
        return

    x3.wait()
    x4b = xfer(x_keep + (1 - b) * b4, b4,
               out_ref.at[pl.ds(x_keep + (1 - b) * b4, b4), :],
               xs.at[5], xr.at[5], p_a)
    x4b.start()
    y3.wait()
    y4b = xfer(y_keep + (1 - a) * b4, b4,
               out_ref.at[pl.ds(y_keep + (1 - a) * b4, b4), :],
               ys.at[5], yr.at[5], p_b)
    y4b.start()
    x4a.wait()
    x4b.wait()
    y4a.wait()
    y4b.wait()


def kernel(ids, E):
    v_per, d = E.shape
    t = ids.shape[0]
    my_pos = lax.axis_index("i")

    local = ids - my_pos * v_per
    mask = (local >= 0) & (local < v_per)

    a = my_pos % 2
    b = my_pos // 2
    k1 = (a + b) % 2
    b2 = t // 4
    x_send = (1 - k1) * b2
    y_send = t // 2 + (1 - b) * b2
    x_keep = k1 * b2
    y_keep = t // 2 + b * b2

    def compact(mask_c, rows_c, local_c):
        s = rows_c.shape[0]
        cs = jnp.cumsum(mask_c.astype(jnp.int32))
        slot = cs - 1
        m = (slot[:, None] == jnp.arange(s, dtype=jnp.int32)[None, :]) \
            & mask_c[:, None]
        p = jnp.sum(jnp.where(m, rows_c[:, None], 0), axis=0)
        ix = jnp.sum(jnp.where(m, local_c[:, None], 0), axis=0)
        return p, ix, cs[-1]

    def region(off, size):
        r = off + jnp.arange(size, dtype=jnp.int32)
        return (lax.dynamic_slice(mask, (off,), (size,)), r,
                lax.dynamic_slice(local, (off,), (size,)))

    mx, rx, lx = region(x_send, b2)
    my_, ry, ly = region(y_send, b2)
    mkx, rkx, lkx = region(x_keep, b2)
    mky, rky, lky = region(y_keep, b2)
    mr = jnp.concatenate([mkx, mky])
    rr = jnp.concatenate([rkx, rky])
    lr = jnp.concatenate([lkx, lky])

    px, ixx, nx = compact(mx, rx, lx)
    py, ixy, ny = compact(my_, ry, ly)
    pr, ixr, nr = compact(mr, rr, lr)
    pos = jnp.concatenate([px, py, pr]).astype(jnp.int32)
    idx = jnp.clip(jnp.concatenate([ixx, ixy, ixr]),
                   0, v_per - 1).astype(jnp.int32)
    cnt = jnp.stack([nx, ny, nr]).astype(jnp.int32)

    return pl.pallas_call(
        _body,
        out_shape=jax.ShapeDtypeStruct((t, d), jnp.float32),
        in_specs=[
            pl.BlockSpec(memory_space=pltpu.SMEM),
            pl.BlockSpec(memory_space=pltpu.SMEM),
            pl.BlockSpec(memory_space=pltpu.SMEM),
            pl.BlockSpec(memory_space=pltpu.MemorySpace.HBM),
        ],
        out_specs=pl.BlockSpec(memory_space=pltpu.VMEM),
        scratch_shapes=[
            pltpu.VMEM((t // 4, d), jnp.float32),
            pltpu.VMEM((t // 8, d), jnp.float32),
            pltpu.VMEM((t // 4, d), jnp.float32),
            pltpu.VMEM((t // 8, d), jnp.float32),
            pltpu.SemaphoreType.DMA((K_DMA,)),
            pltpu.SemaphoreType.DMA((6,)),
            pltpu.SemaphoreType.DMA((6,)),
            pltpu.SemaphoreType.DMA((6,)),
            pltpu.SemaphoreType.DMA((6,)),
        ],
        compiler_params=pltpu.CompilerParams(collective_id=0),
    )(pos, idx, cnt, E)


# device time: 54210 ns/iter; 1.0731x vs baseline; 1.0731x over previous
import os

import jax
import jax.numpy as jnp
from jax import lax
from jax.experimental import pallas as pl
from jax.experimental.pallas import tpu as pltpu

N_DEV = 4
K_DMA = 32

DO_ZERO = os.environ.get("KZ", "1") == "1"
DO_GATHER = os.environ.get("KG", "1") == "1"
DO_COMM = os.environ.get("KC", "1") == "1"
N_ROUNDS = int(os.environ.get("KN", "4"))


def _body(pos_ref, idx_ref, cnt_ref, e_ref, out_ref,
          bx1, bx2, by1, by2, gather_sems, xs, xr, ys, yr):
    t, d = out_ref.shape
    t2 = t // 2
    b2 = t // 4
    b4 = t // 8

    my = lax.axis_index("i")
    a = my % 2
    b = my // 2
    k1 = (a + b) % 2
    p_a = my + 1 - 2 * a
    p_b = 3 - my

    def xfer(src_rows, n_rows, dst, ssem, rsem, peer):
        return pltpu.make_async_remote_copy(
            src_ref=out_ref.at[pl.ds(src_rows, n_rows), :],
            dst_ref=dst,
            send_sem=ssem,
            recv_sem=rsem,
            device_id=(peer,),
            device_id_type=pl.DeviceIdType.MESH,
        )

    def row_dma(j):
        return pltpu.make_async_copy(
            e_ref.at[pl.ds(idx_ref[j], 1), :],
            out_ref.at[pl.ds(pos_ref[j], 1), :],
            gather_sems.at[j % K_DMA],
        )

    def gather(lo, hi):
        def step(j, carry):
            @pl.when(j - lo >= K_DMA)
            def _():
                row_dma(j - K_DMA).wait()
            row_dma(j).start()
            return carry

        lax.fori_loop(lo, hi, step, 0)

        def drain(j, carry):
            row_dma(j).wait()
            return carry

        lax.fori_loop(jnp.maximum(hi - K_DMA, lo), hi, drain, 0)

    barrier_sem = pltpu.get_barrier_semaphore()
    for nbr in [p_a, p_b]:
        pl.semaphore_signal(
            barrier_sem, inc=1,
            device_id=(nbr,), device_id_type=pl.DeviceIdType.MESH,
        )
    pl.semaphore_wait(barrier_sem, 2)


    x_send = (1 - k1) * b2
    y_send = t2 + (1 - b) * b2
    x_keep = k1 * b2
    y_keep = t2 + b * b2
    x_q_keep = x_keep + b * b4
    y_q_keep = y_keep + a * b4

    def zero_half(off):
        out_ref[pl.ds(off, b2), :] = jnp.zeros((b2, d), jnp.float32)

    def add_quarter(base, q, buf):
        out_ref[pl.ds(base + q, b4), :] = (
            out_ref[pl.ds(base + q, b4), :] + buf[pl.ds(q, b4), :]
        )

    if DO_ZERO:
        zero_half(x_send)
    if DO_GATHER:
        gather(x_send, x_send + cnt_ref[1 - k1])
    if not DO_COMM:
        if DO_ZERO:
            zero_half(y_send)
            zero_half(x_keep)
            zero_half(y_keep)
        if DO_GATHER:
            gather(y_send, y_send + cnt_ref[3 - b])
            gather(x_keep, x_keep + cnt_ref[k1])
            gather(y_keep, y_keep + cnt_ref[2 + b])
        return
    x1a = xfer(x_send + (1 - b) * b4, b4,
               bx1.at[pl.ds((1 - b) * b4, b4), :], xs.at[0], xr.at[0], p_a)
    x1b = xfer(x_send + b * b4, b4,
               bx1.at[pl.ds(b * b4, b4), :], xs.at[1], xr.at[1], p_a)
    x1a.start()
    x1b.start()
    if DO_ZERO:
        zero_half(y_send)
    if DO_GATHER:
        gather(y_send, y_send + cnt_ref[3 - b])
    y1a = xfer(y_send + a * b4, b4,
               by1.at[pl.ds(a * b4, b4), :], ys.at[0], yr.at[0], p_b)
    y1b = xfer(y_send + (1 - a) * b4, b4,
               by1.at[pl.ds((1 - a) * b4, b4), :], ys.at[1], yr.at[1], p_b)
    y1a.start()
    y1b.start()

    if DO_ZERO:
        zero_half(x_keep)
        zero_half(y_keep)
    if DO_GATHER:
        gather(x_keep, x_keep + cnt_ref[k1])
        gather(y_keep, y_keep + cnt_ref[2 + b])

    if N_ROUNDS <= 1:
        x1a.wait()
        x1b.wait()
        y1a.wait()
        y1b.wait()
        return

    x1a.wait()
    add_quarter(x_keep, (1 - b) * b4, bx1)
    x2 = xfer(x_keep + (1 - b) * b4, b4, bx2, xs.at[2], xr.at[2], p_b)
    x2.start()
    y1a.wait()
    add_quarter(y_keep, (1 - a) * b4, by1)
    y2 = xfer(y_keep + (1 - a) * b4, b4, by2, ys.at[2], yr.at[2], p_a)
    y2.start()
    x1b.wait()
    add_quarter(x_keep, b * b4, bx1)
    y1b.wait()
    add_quarter(y_keep, a * b4, by1)

    if N_ROUNDS <= 2:
        x2.wait()
        out_ref[pl.ds(x_q_keep, b4), :] = (
            out_ref[pl.ds(x_q_keep, b4), :] + bx2[:, :]
        )
        y2.wait()
        out_ref[pl.ds(y_q_keep, b4), :] = (
            out_ref[pl.ds(y_q_keep, b4), :] + by2[:, :]
        )
        return

    x2.wait()
    out_ref[pl.ds(x_q_keep, b4), :] = (
        out_ref[pl.ds(x_q_keep, b4), :] + bx2[:, :]
    )
    x3 = xfer(x_q_keep, b4, out_ref.at[pl.ds(x_q_keep, b4), :],
              xs.at[3], xr.at[3], p_b)
    x3.start()
    x4a = xfer(x_q_keep, b4, out_ref.at[pl.ds(x_q_keep, b4), :],
               xs.at[4], xr.at[4], p_a)
    x4a.start()
    y2.wait()
    out_ref[pl.ds(y_q_keep, b4), :] = (
        out_ref[pl.ds(y_q_keep, b4), :] + by2[:, :]
    )
    y3 = xfer(y_q_keep, b4, out_ref.at[pl.ds(y_q_keep, b4), :],
              ys.at[3], yr.at[3], p_a)
    y3.start()
    y4a = xfer(y_q_keep, b4, out_ref.at[pl.ds(y_q_keep, b4), :],
               ys.at[4], yr.at[4], p_b)
    y4a.start()

    if N_ROUNDS <= 3:
        x3.wait()
        y3.wait()
        x4a.wait()
        y4a.wait()
        return

    x3.wait()
    x4b = xfer(x_keep + (1 - b) * b4, b4,
               out_ref.at[pl.ds(x_keep + (1 - b) * b4, b4), :],
               xs.at[5], xr.at[5], p_a)
    x4b.start()
    y3.wait()
    y4b = xfer(y_keep + (1 - a) * b4, b4,
               out_ref.at[pl.ds(y_keep + (1 - a) * b4, b4), :],
               ys.at[5], yr.at[5], p_b)
    y4b.start()
    x4a.wait()
    x4b.wait()
    y4a.wait()
    y4b.wait()


def kernel(ids, E):
    v_per, d = E.shape
    t = ids.shape[0]
    my_pos = lax.axis_index("i")

    local = ids - my_pos * v_per
    mask = (local >= 0) & (local < v_per)

    a = my_pos % 2
    b = my_pos // 2
    k1 = (a + b) % 2
    b2 = t // 4
    x_send = (1 - k1) * b2
    y_send = t // 2 + (1 - b) * b2
    mask2 = mask.reshape(4, b2)
    local2 = local.reshape(4, b2)
    rows2 = jnp.arange(t, dtype=jnp.int32).reshape(4, b2)
    cs = jnp.cumsum(mask2.astype(jnp.int32), axis=1)
    m = (cs[:, :, None] - 1 == jnp.arange(b2, dtype=jnp.int32)[None, None, :]
         ) & mask2[:, :, None]
    pos = jnp.sum(jnp.where(m, rows2[:, :, None], 0), axis=1
                  ).reshape(t).astype(jnp.int32)
    idx = jnp.clip(jnp.sum(jnp.where(m, local2[:, :, None], 0), axis=1),
                   0, v_per - 1).reshape(t).astype(jnp.int32)
    cnt = cs[:, -1].astype(jnp.int32)

    return pl.pallas_call(
        _body,
        out_shape=jax.ShapeDtypeStruct((t, d), jnp.float32),
        in_specs=[
            pl.BlockSpec(memory_space=pltpu.SMEM),
            pl.BlockSpec(memory_space=pltpu.SMEM),
            pl.BlockSpec(memory_space=pltpu.SMEM),
            pl.BlockSpec(memory_space=pltpu.MemorySpace.HBM),
        ],
        out_specs=pl.BlockSpec(memory_space=pltpu.VMEM),
        scratch_shapes=[
            pltpu.VMEM((t // 4, d), jnp.float32),
            pltpu.VMEM((t // 8, d), jnp.float32),
            pltpu.VMEM((t // 4, d), jnp.float32),
            pltpu.VMEM((t // 8, d), jnp.float32),
            pltpu.SemaphoreType.DMA((K_DMA,)),
            pltpu.SemaphoreType.DMA((6,)),
            pltpu.SemaphoreType.DMA((6,)),
            pltpu.SemaphoreType.DMA((6,)),
            pltpu.SemaphoreType.DMA((6,)),
        ],
        compiler_params=pltpu.CompilerParams(collective_id=0),
    )(pos, idx, cnt, E)


# device time: 49778 ns/iter; 1.1686x vs baseline; 1.0890x over previous
import os

import jax
import jax.numpy as jnp
from jax import lax
from jax.experimental import pallas as pl
from jax.experimental.pallas import tpu as pltpu

N_DEV = 4
K_DMA = 32

DO_ZERO = os.environ.get("KZ", "1") == "1"
DO_GATHER = os.environ.get("KG", "1") == "1"
DO_COMM = os.environ.get("KC", "1") == "1"
N_ROUNDS = int(os.environ.get("KN", "4"))

S_1A0, S_1A1, S_1B, S_2A, S_2B, S_3A, S_3B, S_4A0, S_4A1, S_4B0, S_4B1 \
    = range(11)


def _body(pos_ref, idx_ref, cnt_ref, e_ref, out_ref,
          bx1, bx2, by1, by2, gather_sems, xs, xr, ys, yr):
    t, d = out_ref.shape
    t2 = t // 2
    b2 = t // 4
    b4 = t // 8
    b8 = t // 16

    my = lax.axis_index("i")
    a = my % 2
    b = my // 2
    k1 = (a + b) % 2
    p_a = my + 1 - 2 * a
    p_b = 3 - my

    def xfer(src_rows, n_rows, dst, ssem, rsem, peer):
        return pltpu.make_async_remote_copy(
            src_ref=out_ref.at[pl.ds(src_rows, n_rows), :],
            dst_ref=dst,
            send_sem=ssem,
            recv_sem=rsem,
            device_id=(peer,),
            device_id_type=pl.DeviceIdType.MESH,
        )

    def row_dma(j):
        return pltpu.make_async_copy(
            e_ref.at[pl.ds(idx_ref[j], 1), :],
            out_ref.at[pl.ds(pos_ref[j], 1), :],
            gather_sems.at[j % K_DMA],
        )

    def gather(lo, hi):
        def step(j, carry):
            @pl.when(j - lo >= K_DMA)
            def _():
                row_dma(j - K_DMA).wait()
            row_dma(j).start()
            return carry

        lax.fori_loop(lo, hi, step, 0)

        def drain(j, carry):
            row_dma(j).wait()
            return carry

        lax.fori_loop(jnp.maximum(hi - K_DMA, lo), hi, drain, 0)

    def prep_region(r):
        if DO_ZERO:
            out_ref[pl.ds(r * b4, b4), :] = jnp.zeros((b4, d), jnp.float32)
        if DO_GATHER:
            gather(r * b4, r * b4 + cnt_ref[r])

    barrier_sem = pltpu.get_barrier_semaphore()
    for nbr in [p_a, p_b]:
        pl.semaphore_signal(
            barrier_sem, inc=1,
            device_id=(nbr,), device_id_type=pl.DeviceIdType.MESH,
        )
    pl.semaphore_wait(barrier_sem, 2)

    x_send = (1 - k1) * b2
    y_send = t2 + (1 - b) * b2
    x_keep = k1 * b2
    y_keep = t2 + b * b2
    x_fwd = x_keep + (1 - b) * b4
    y_fwd = y_keep + (1 - a) * b4
    x_q_keep = x_keep + b * b4
    y_q_keep = y_keep + a * b4

    def add_rows(dst_off, buf, buf_off, n_rows):
        out_ref[pl.ds(dst_off, n_rows), :] = (
            out_ref[pl.ds(dst_off, n_rows), :]
            + buf[pl.ds(buf_off, n_rows), :]
        )

    prep_region(2 * (1 - k1) + (1 - b))
    if not DO_COMM:
        prep_region(2 * (1 - k1) + b)
        prep_region(4 + 2 * (1 - b) + a)
        prep_region(4 + 2 * (1 - b) + (1 - a))
        prep_region(2 * k1 + (1 - b))
        prep_region(2 * k1 + b)
        prep_region(4 + 2 * b + (1 - a))
        prep_region(4 + 2 * b + a)
        return
    xf0 = x_send + (1 - b) * b4
    x1a0 = xfer(xf0, b8, bx1.at[pl.ds((1 - b) * b4, b8), :],
                xs.at[S_1A0], xr.at[S_1A0], p_a)
    x1a1 = xfer(xf0 + b8, b8, bx1.at[pl.ds((1 - b) * b4 + b8, b8), :],
                xs.at[S_1A1], xr.at[S_1A1], p_a)
    x1a0.start()
    x1a1.start()
    prep_region(2 * (1 - k1) + b)
    x1b = xfer(x_send + b * b4, b4, bx1.at[pl.ds(b * b4, b4), :],
               xs.at[S_1B], xr.at[S_1B], p_a)
    x1b.start()
    prep_region(4 + 2 * (1 - b) + a)
    yf0 = y_send + a * b4
    y1a0 = xfer(yf0, b8, by1.at[pl.ds(a * b4, b8), :],
                ys.at[S_1A0], yr.at[S_1A0], p_b)
    y1a1 = xfer(yf0 + b8, b8, by1.at[pl.ds(a * b4 + b8, b8), :],
                ys.at[S_1A1], yr.at[S_1A1], p_b)
    y1a0.start()
    y1a1.start()
    prep_region(4 + 2 * (1 - b) + (1 - a))
    y1b = xfer(y_send + (1 - a) * b4, b4, by1.at[pl.ds((1 - a) * b4, b4), :],
               ys.at[S_1B], yr.at[S_1B], p_b)
    y1b.start()

    prep_region(2 * k1 + (1 - b))
    prep_region(4 + 2 * b + (1 - a))
    prep_region(2 * k1 + b)
    prep_region(4 + 2 * b + a)

    if N_ROUNDS <= 1:
        for r in (x1a0, x1a1, x1b, y1a0, y1a1, y1b):
            r.wait()
        return

    x1a0.wait()
    add_rows(x_fwd, bx1, (1 - b) * b4, b8)
    x2a = xfer(x_fwd, b8, bx2.at[pl.ds(0, b8), :],
               xs.at[S_2A], xr.at[S_2A], p_b)
    x2a.start()
    y1a0.wait()
    add_rows(y_fwd, by1, (1 - a) * b4, b8)
    y2a = xfer(y_fwd, b8, by2.at[pl.ds(0, b8), :],
               ys.at[S_2A], yr.at[S_2A], p_a)
    y2a.start()
    x1a1.wait()
    add_rows(x_fwd + b8, bx1, (1 - b) * b4 + b8, b8)
    x2b = xfer(x_fwd + b8, b8, bx2.at[pl.ds(b8, b8), :],
               xs.at[S_2B], xr.at[S_2B], p_b)
    x2b.start()
    y1a1.wait()
    add_rows(y_fwd + b8, by1, (1 - a) * b4 + b8, b8)
    y2b = xfer(y_fwd + b8, b8, by2.at[pl.ds(b8, b8), :],
               ys.at[S_2B], yr.at[S_2B], p_a)
    y2b.start()
    x1b.wait()
    add_rows(x_keep + b * b4, bx1, b * b4, b4)
    y1b.wait()
    add_rows(y_keep + a * b4, by1, a * b4, b4)

    if N_ROUNDS <= 2:
        x2a.wait()
        x2b.wait()
        add_rows(x_q_keep, bx2, 0, b4)
        y2a.wait()
        y2b.wait()
        add_rows(y_q_keep, by2, 0, b4)
        return

    x2a.wait()
    add_rows(x_q_keep, bx2, 0, b8)
    x3a = xfer(x_q_keep, b8, out_ref.at[pl.ds(x_q_keep, b8), :],
               xs.at[S_3A], xr.at[S_3A], p_b)
    x3a.start()
    x4a0 = xfer(x_q_keep, b8, out_ref.at[pl.ds(x_q_keep, b8), :],
                xs.at[S_4A0], xr.at[S_4A0], p_a)
    x4a0.start()
    y2a.wait()
    add_rows(y_q_keep, by2, 0, b8)
    y3a = xfer(y_q_keep, b8, out_ref.at[pl.ds(y_q_keep, b8), :],
               ys.at[S_3A], yr.at[S_3A], p_a)
    y3a.start()
    y4a0 = xfer(y_q_keep, b8, out_ref.at[pl.ds(y_q_keep, b8), :],
                ys.at[S_4A0], yr.at[S_4A0], p_b)
    y4a0.start()
    x2b.wait()
    add_rows(x_q_keep + b8, bx2, b8, b8)
    x3b = xfer(x_q_keep + b8, b8, out_ref.at[pl.ds(x_q_keep + b8, b8), :],
               xs.at[S_3B], xr.at[S_3B], p_b)
    x3b.start()
    x4a1 = xfer(x_q_keep + b8, b8, out_ref.at[pl.ds(x_q_keep + b8, b8), :],
                xs.at[S_4A1], xr.at[S_4A1], p_a)
    x4a1.start()
    y2b.wait()
    add_rows(y_q_keep + b8, by2, b8, b8)
    y3b = xfer(y_q_keep + b8, b8, out_ref.at[pl.ds(y_q_keep + b8, b8), :],
               ys.at[S_3B], yr.at[S_3B], p_a)
    y3b.start()
    y4a1 = xfer(y_q_keep + b8, b8, out_ref.at[pl.ds(y_q_keep + b8, b8), :],
                ys.at[S_4A1], yr.at[S_4A1], p_b)
    y4a1.start()

    if N_ROUNDS <= 3:
        for r in (x3a, x3b, y3a, y3b, x4a0, x4a1, y4a0, y4a1):
            r.wait()
        return

    x3a.wait()
    x4b0 = xfer(x_fwd, b8, out_ref.at[pl.ds(x_fwd, b8), :],
                xs.at[S_4B0], xr.at[S_4B0], p_a)
    x4b0.start()
    y3a.wait()
    y4b0 = xfer(y_fwd, b8, out_ref.at[pl.ds(y_fwd, b8), :],
                ys.at[S_4B0], yr.at[S_4B0], p_b)
    y4b0.start()
    x3b.wait()
    x4b1 = xfer(x_fwd + b8, b8, out_ref.at[pl.ds(x_fwd + b8, b8), :],
                xs.at[S_4B1], xr.at[S_4B1], p_a)
    x4b1.start()
    y3b.wait()
    y4b1 = xfer(y_fwd + b8, b8, out_ref.at[pl.ds(y_fwd + b8, b8), :],
                ys.at[S_4B1], yr.at[S_4B1], p_b)
    y4b1.start()

    for r in (x4a0, x4a1, x4b0, x4b1, y4a0, y4a1, y4b0, y4b1):
        r.wait()


def kernel(ids, E):
    v_per, d = E.shape
    t = ids.shape[0]
    my_pos = lax.axis_index("i")

    local = ids - my_pos * v_per
    mask = (local >= 0) & (local < v_per)

    b4 = t // 8
    mask2 = mask.reshape(8, b4)
    local2 = local.reshape(8, b4)
    rows2 = jnp.arange(t, dtype=jnp.int32).reshape(8, b4)
    cs = jnp.cumsum(mask2.astype(jnp.int32), axis=1)
    m = (cs[:, :, None] - 1 == jnp.arange(b4, dtype=jnp.int32)[None, None, :]
         ) & mask2[:, :, None]
    pos = jnp.sum(jnp.where(m, rows2[:, :, None], 0), axis=1
                  ).reshape(t).astype(jnp.int32)
    idx = jnp.clip(jnp.sum(jnp.where(m, local2[:, :, None], 0), axis=1),
                   0, v_per - 1).reshape(t).astype(jnp.int32)
    cnt = cs[:, -1].astype(jnp.int32)

    return pl.pallas_call(
        _body,
        out_shape=jax.ShapeDtypeStruct((t, d), jnp.float32),
        in_specs=[
            pl.BlockSpec(memory_space=pltpu.SMEM),
            pl.BlockSpec(memory_space=pltpu.SMEM),
            pl.BlockSpec(memory_space=pltpu.SMEM),
            pl.BlockSpec(memory_space=pltpu.MemorySpace.HBM),
        ],
        out_specs=pl.BlockSpec(memory_space=pltpu.VMEM),
        scratch_shapes=[
            pltpu.VMEM((t // 4, d), jnp.float32),
            pltpu.VMEM((t // 8, d), jnp.float32),
            pltpu.VMEM((t // 4, d), jnp.float32),
            pltpu.VMEM((t // 8, d), jnp.float32),
            pltpu.SemaphoreType.DMA((K_DMA,)),
            pltpu.SemaphoreType.DMA((11,)),
            pltpu.SemaphoreType.DMA((11,)),
            pltpu.SemaphoreType.DMA((11,)),
            pltpu.SemaphoreType.DMA((11,)),
        ],
        compiler_params=pltpu.CompilerParams(collective_id=0),
    )(pos, idx, cnt, E)


# device time: 49447 ns/iter; 1.1765x vs baseline; 1.0067x over previous
import os

import jax
import jax.numpy as jnp
from jax import lax
from jax.experimental import pallas as pl
from jax.experimental.pallas import tpu as pltpu

N_DEV = 4
K_DMA = 16

DO_ZERO = os.environ.get("KZ", "1") == "1"
DO_GATHER = os.environ.get("KG", "1") == "1"
DO_COMM = os.environ.get("KC", "1") == "1"
N_ROUNDS = int(os.environ.get("KN", "4"))

S_1A0, S_1A1, S_1B, S_2A, S_2B, S_3A, S_3B, S_4A0, S_4A1, S_4B0, S_4B1 \
    = range(11)


def _body(pos_ref, idx_ref, cnt_ref, e_ref, out_ref,
          bx1, bx2, by1, by2, gather_sems, xs, xr, ys, yr):
    t, d = out_ref.shape
    t2 = t // 2
    b2 = t // 4
    b4 = t // 8
    b8 = t // 16

    my = lax.axis_index("i")
    a = my % 2
    b = my // 2
    k1 = (a + b) % 2
    p_a = my + 1 - 2 * a
    p_b = 3 - my

    def xfer(src_rows, n_rows, dst, ssem, rsem, peer):
        return pltpu.make_async_remote_copy(
            src_ref=out_ref.at[pl.ds(src_rows, n_rows), :],
            dst_ref=dst,
            send_sem=ssem,
            recv_sem=rsem,
            device_id=(peer,),
            device_id_type=pl.DeviceIdType.MESH,
        )

    def row_dma(r, j):
        return pltpu.make_async_copy(
            e_ref.at[pl.ds(idx_ref[j], 1), :],
            out_ref.at[pl.ds(pos_ref[j], 1), :],
            gather_sems.at[r, (j - r * b4) % K_DMA],
        )

    def zero_region(r):
        if DO_ZERO:
            out_ref[pl.ds(r * b4, b4), :] = jnp.zeros((b4, d), jnp.float32)

    def issue_region(r):
        if not DO_GATHER:
            return
        lo = r * b4
        hi = lo + cnt_ref[r]

        def step(j, carry):
            @pl.when(j - lo >= K_DMA)
            def _():
                row_dma(r, j - K_DMA).wait()
            row_dma(r, j).start()
            return carry

        lax.fori_loop(lo, hi, step, 0)

    def drain_region(r):
        if not DO_GATHER:
            return
        lo = r * b4
        hi = lo + cnt_ref[r]

        def drain(j, carry):
            row_dma(r, j).wait()
            return carry

        lax.fori_loop(jnp.maximum(hi - K_DMA, lo), hi, drain, 0)

    barrier_sem = pltpu.get_barrier_semaphore()
    for nbr in [p_a, p_b]:
        pl.semaphore_signal(
            barrier_sem, inc=1,
            device_id=(nbr,), device_id_type=pl.DeviceIdType.MESH,
        )
    pl.semaphore_wait(barrier_sem, 2)

    x_send = (1 - k1) * b2
    y_send = t2 + (1 - b) * b2
    x_keep = k1 * b2
    y_keep = t2 + b * b2
    x_fwd = x_keep + (1 - b) * b4
    y_fwd = y_keep + (1 - a) * b4
    x_q_keep = x_keep + b * b4
    y_q_keep = y_keep + a * b4

    def add_rows(dst_off, buf, buf_off, n_rows):
        out_ref[pl.ds(dst_off, n_rows), :] = (
            out_ref[pl.ds(dst_off, n_rows), :]
            + buf[pl.ds(buf_off, n_rows), :]
        )

    r_xsf = 2 * (1 - k1) + (1 - b)
    r_xso = 2 * (1 - k1) + b
    r_ysf = 4 + 2 * (1 - b) + a
    r_yso = 4 + 2 * (1 - b) + (1 - a)
    r_xkf = 2 * k1 + (1 - b)
    r_xko = 2 * k1 + b
    r_ykf = 4 + 2 * b + (1 - a)
    r_yko = 4 + 2 * b + a

    if not DO_COMM:
        for r in (r_xsf, r_xso, r_ysf, r_yso, r_xkf, r_xko, r_ykf, r_yko):
            zero_region(r)
            issue_region(r)
            drain_region(r)
        return

    zero_region(r_xsf)
    issue_region(r_xsf)
    zero_region(r_ysf)
    drain_region(r_xsf)
    xf0 = x_send + (1 - b) * b4
    x1a0 = xfer(xf0, b8, bx1.at[pl.ds((1 - b) * b4, b8), :],
                xs.at[S_1A0], xr.at[S_1A0], p_a)
    x1a1 = xfer(xf0 + b8, b8, bx1.at[pl.ds((1 - b) * b4 + b8, b8), :],
                xs.at[S_1A1], xr.at[S_1A1], p_a)
    x1a0.start()
    x1a1.start()
    issue_region(r_ysf)
    zero_region(r_xso)
    drain_region(r_ysf)
    yf0 = y_send + a * b4
    y1a0 = xfer(yf0, b8, by1.at[pl.ds(a * b4, b8), :],
                ys.at[S_1A0], yr.at[S_1A0], p_b)
    y1a1 = xfer(yf0 + b8, b8, by1.at[pl.ds(a * b4 + b8, b8), :],
                ys.at[S_1A1], yr.at[S_1A1], p_b)
    y1a0.start()
    y1a1.start()
    issue_region(r_xso)
    zero_region(r_yso)
    drain_region(r_xso)
    x1b = xfer(x_send + b * b4, b4, bx1.at[pl.ds(b * b4, b4), :],
               xs.at[S_1B], xr.at[S_1B], p_a)
    x1b.start()
    issue_region(r_yso)
    zero_region(r_xkf)
    drain_region(r_yso)
    y1b = xfer(y_send + (1 - a) * b4, b4, by1.at[pl.ds((1 - a) * b4, b4), :],
               ys.at[S_1B], yr.at[S_1B], p_b)
    y1b.start()

    issue_region(r_xkf)
    zero_region(r_ykf)
    issue_region(r_ykf)
    zero_region(r_xko)
    issue_region(r_xko)
    zero_region(r_yko)
    issue_region(r_yko)

    if N_ROUNDS <= 1:
        for rg in (r_xkf, r_ykf, r_xko, r_yko):
            drain_region(rg)
        for r in (x1a0, x1a1, x1b, y1a0, y1a1, y1b):
            r.wait()
        return

    drain_region(r_xkf)
    x1a0.wait()
    add_rows(x_fwd, bx1, (1 - b) * b4, b8)
    x2a = xfer(x_fwd, b8, bx2.at[pl.ds(0, b8), :],
               xs.at[S_2A], xr.at[S_2A], p_b)
    x2a.start()
    drain_region(r_ykf)
    y1a0.wait()
    add_rows(y_fwd, by1, (1 - a) * b4, b8)
    y2a = xfer(y_fwd, b8, by2.at[pl.ds(0, b8), :],
               ys.at[S_2A], yr.at[S_2A], p_a)
    y2a.start()
    x1a1.wait()
    add_rows(x_fwd + b8, bx1, (1 - b) * b4 + b8, b8)
    x2b = xfer(x_fwd + b8, b8, bx2.at[pl.ds(b8, b8), :],
               xs.at[S_2B], xr.at[S_2B], p_b)
    x2b.start()
    y1a1.wait()
    add_rows(y_fwd + b8, by1, (1 - a) * b4 + b8, b8)
    y2b = xfer(y_fwd + b8, b8, by2.at[pl.ds(b8, b8), :],
               ys.at[S_2B], yr.at[S_2B], p_a)
    y2b.start()
    drain_region(r_xko)
    x1b.wait()
    add_rows(x_keep + b * b4, bx1, b * b4, b4)
    drain_region(r_yko)
    y1b.wait()
    add_rows(y_keep + a * b4, by1, a * b4, b4)

    if N_ROUNDS <= 2:
        x2a.wait()
        x2b.wait()
        add_rows(x_q_keep, bx2, 0, b4)
        y2a.wait()
        y2b.wait()
        add_rows(y_q_keep, by2, 0, b4)
        return

    x2a.wait()
    add_rows(x_q_keep, bx2, 0, b8)
    x3a = xfer(x_q_keep, b8, out_ref.at[pl.ds(x_q_keep, b8), :],
               xs.at[S_3A], xr.at[S_3A], p_b)
    x3a.start()
    x4a0 = xfer(x_q_keep, b8, out_ref.at[pl.ds(x_q_keep, b8), :],
                xs.at[S_4A0], xr.at[S_4A0], p_a)
    x4a0.start()
    y2a.wait()
    add_rows(y_q_keep, by2, 0, b8)
    y3a = xfer(y_q_keep, b8, out_ref.at[pl.ds(y_q_keep, b8), :],
               ys.at[S_3A], yr.at[S_3A], p_a)
    y3a.start()
    y4a0 = xfer(y_q_keep, b8, out_ref.at[pl.ds(y_q_keep, b8), :],
                ys.at[S_4A0], yr.at[S_4A0], p_b)
    y4a0.start()
    x2b.wait()
    add_rows(x_q_keep + b8, bx2, b8, b8)
    x3b = xfer(x_q_keep + b8, b8, out_ref.at[pl.ds(x_q_keep + b8, b8), :],
               xs.at[S_3B], xr.at[S_3B], p_b)
    x3b.start()
    x4a1 = xfer(x_q_keep + b8, b8, out_ref.at[pl.ds(x_q_keep + b8, b8), :],
                xs.at[S_4A1], xr.at[S_4A1], p_a)
    x4a1.start()
    y2b.wait()
    add_rows(y_q_keep + b8, by2, b8, b8)
    y3b = xfer(y_q_keep + b8, b8, out_ref.at[pl.ds(y_q_keep + b8, b8), :],
               ys.at[S_3B], yr.at[S_3B], p_a)
    y3b.start()
    y4a1 = xfer(y_q_keep + b8, b8, out_ref.at[pl.ds(y_q_keep + b8, b8), :],
                ys.at[S_4A1], yr.at[S_4A1], p_b)
    y4a1.start()

    if N_ROUNDS <= 3:
        for r in (x3a, x3b, y3a, y3b, x4a0, x4a1, y4a0, y4a1):
            r.wait()
        return

    x3a.wait()
    x4b0 = xfer(x_fwd, b8, out_ref.at[pl.ds(x_fwd, b8), :],
                xs.at[S_4B0], xr.at[S_4B0], p_a)
    x4b0.start()
    y3a.wait()
    y4b0 = xfer(y_fwd, b8, out_ref.at[pl.ds(y_fwd, b8), :],
                ys.at[S_4B0], yr.at[S_4B0], p_b)
    y4b0.start()
    x3b.wait()
    x4b1 = xfer(x_fwd + b8, b8, out_ref.at[pl.ds(x_fwd + b8, b8), :],
                xs.at[S_4B1], xr.at[S_4B1], p_a)
    x4b1.start()
    y3b.wait()
    y4b1 = xfer(y_fwd + b8, b8, out_ref.at[pl.ds(y_fwd + b8, b8), :],
                ys.at[S_4B1], yr.at[S_4B1], p_b)
    y4b1.start()

    for r in (x4a0, x4a1, x4b0, x4b1, y4a0, y4a1, y4b0, y4b1):
        r.wait()


def kernel(ids, E):
    v_per, d = E.shape
    t = ids.shape[0]
    my_pos = lax.axis_index("i")

    local = ids - my_pos * v_per
    mask = (local >= 0) & (local < v_per)

    b4 = t // 8
    mask2 = mask.reshape(8, b4)
    local2 = local.reshape(8, b4)
    rows2 = jnp.arange(t, dtype=jnp.int32).reshape(8, b4)
    cs = jnp.cumsum(mask2.astype(jnp.int32), axis=1)
    m = (cs[:, :, None] - 1 == jnp.arange(b4, dtype=jnp.int32)[None, None, :]
         ) & mask2[:, :, None]
    pos = jnp.sum(jnp.where(m, rows2[:, :, None], 0), axis=1
                  ).reshape(t).astype(jnp.int32)
    idx = jnp.clip(jnp.sum(jnp.where(m, local2[:, :, None], 0), axis=1),
                   0, v_per - 1).reshape(t).astype(jnp.int32)
    cnt = cs[:, -1].astype(jnp.int32)

    return pl.pallas_call(
        _body,
        out_shape=jax.ShapeDtypeStruct((t, d), jnp.float32),
        in_specs=[
            pl.BlockSpec(memory_space=pltpu.SMEM),
            pl.BlockSpec(memory_space=pltpu.SMEM),
            pl.BlockSpec(memory_space=pltpu.SMEM),
            pl.BlockSpec(memory_space=pltpu.MemorySpace.HBM),
        ],
        out_specs=pl.BlockSpec(memory_space=pltpu.VMEM),
        scratch_shapes=[
            pltpu.VMEM((t // 4, d), jnp.float32),
            pltpu.VMEM((t // 8, d), jnp.float32),
            pltpu.VMEM((t // 4, d), jnp.float32),
            pltpu.VMEM((t // 8, d), jnp.float32),
            pltpu.SemaphoreType.DMA((8, K_DMA)),
            pltpu.SemaphoreType.DMA((11,)),
            pltpu.SemaphoreType.DMA((11,)),
            pltpu.SemaphoreType.DMA((11,)),
            pltpu.SemaphoreType.DMA((11,)),
        ],
        compiler_params=pltpu.CompilerParams(collective_id=0),
    )(pos, idx, cnt, E)
